# baseline (speedup 1.0000x reference)
import jax
import jax.numpy as jnp
from jax import lax
from jax.experimental import pallas as pl
from jax.experimental.pallas import tpu as pltpu
from jax.experimental.pallas import tpu_sc as plsc

VOCAB = 1000000
MAX_SEQ = 2048
DIM = 64
BATCH = 32

NUM_CORES = 2
CHUNK = 256
NUM_CHUNKS = MAX_SEQ // CHUNK
IDX_ROWS = CHUNK // 128
LANES = 16
GROUPS = CHUNK // LANES
VECS = DIM // LANES


def _sc_body(x_hbm, tok_hbm, pos_hbm, out_hbm,
             idx_v, pair_v, rows2_v, out_v, pos_v, sem):
    b = lax.axis_index("s") * NUM_CORES + lax.axis_index("c")

    def chunk_body(c, _):
        off = pl.multiple_of(c * CHUNK, CHUNK)
        pltpu.sync_copy(x_hbm.at[b, pl.ds(off, CHUNK)], idx_v)

        def shift(g, _):
            base = pl.multiple_of(g * LANES, LANES)
            vec = idx_v[pl.ds(base, LANES)]
            pair_v[g // 8, pl.ds(pl.multiple_of((g % 8) * LANES, LANES),
                                 LANES)] = lax.shift_right_logical(vec, 1)
            return 0

        lax.fori_loop(0, GROUPS, shift, 0)

        copies = [
            pltpu.async_copy(
                tok_hbm.at[pair_v.at[q]],
                rows2_v.at[pl.ds(q * 128, 128), :], sem)
            for q in range(IDX_ROWS)
        ]
        pltpu.sync_copy(pos_hbm.at[pl.ds(off, CHUNK), :], pos_v)
        for cp in copies:
            cp.wait()

        def pick_add(g, _):
            base = pl.multiple_of(g * LANES, LANES)
            half = (idx_v[pl.ds(base, LANES)] & 1) * DIM
            for k in range(LANES):
                j = base + k
                h = half[k]
                for v in range(VECS):
                    s = pl.ds(v * LANES, LANES)
                    out_v[j, s] = rows2_v[j, pl.ds(h + v * LANES, LANES)] \
                        + pos_v[j, s]
            return 0

        lax.fori_loop(0, GROUPS, pick_add, 0)
        pltpu.sync_copy(out_v, out_hbm.at[pl.ds(b * MAX_SEQ + off, CHUNK), :])
        return 0

    lax.fori_loop(0, NUM_CHUNKS, chunk_body, 0)


@jax.jit
def kernel(x, token_table, pos_table):
    tok2 = token_table.reshape(VOCAB // 2, 2 * DIM)
    mesh = plsc.VectorSubcoreMesh(core_axis_name="c", subcore_axis_name="s")
    out = pl.kernel(
        _sc_body,
        out_type=jax.ShapeDtypeStruct((BATCH * MAX_SEQ, DIM), jnp.float32),
        mesh=mesh,
        scratch_types=[
            pltpu.VMEM((CHUNK,), jnp.int32),
            pltpu.VMEM((IDX_ROWS, 128), jnp.int32),
            pltpu.VMEM((CHUNK, 2 * DIM), jnp.float32),
            pltpu.VMEM((CHUNK, DIM), jnp.float32),
            pltpu.VMEM((CHUNK, DIM), jnp.float32),
            pltpu.SemaphoreType.DMA,
        ],
    )(x, tok2, pos_table)
    return out.reshape(BATCH, MAX_SEQ, DIM)


# speedup vs baseline: 1.6483x; 1.6483x over previous
import functools

import jax
import jax.numpy as jnp
from jax import lax
from jax.experimental import pallas as pl
from jax.experimental.pallas import tpu as pltpu
from jax.experimental.pallas import tpu_sc as plsc
from jax.experimental import layout as jex_layout

VOCAB = 1000000
MAX_SEQ = 2048
DIM = 64
BATCH = 32

NUM_CORES = 2
NUM_SUBCORES = 16
CHUNK = 512
NUM_CHUNKS = MAX_SEQ // CHUNK
LANES = 16
VECS_PER_ROW = DIM // LANES


def _sc_body(x_hbm, tok_hbm, pos_hbm, out_hbm, idx_v, rows_v, pos_v, sem):
    wid = lax.axis_index("s") * NUM_CORES + lax.axis_index("c")
    base = wid * MAX_SEQ

    pltpu.sync_copy(x_hbm.at[pl.ds(base, MAX_SEQ)], idx_v)

    for c in range(NUM_CHUNKS):
        gather = pltpu.async_copy(
            tok_hbm.at[idx_v.at[pl.ds(c * CHUNK, CHUNK)]], rows_v, sem)
        pltpu.sync_copy(pos_hbm.at[pl.ds(c * CHUNK, CHUNK)], pos_v)
        gather.wait()

        def add_row(i, _):
            for j in range(VECS_PER_ROW):
                s = pl.ds(j * LANES, LANES)
                rows_v[i, s] = rows_v[i, s] + pos_v[i, s]
            return 0

        lax.fori_loop(0, CHUNK, add_row, 0)

        pltpu.sync_copy(rows_v, out_hbm.at[pl.ds(base + c * CHUNK, CHUNK)])


@jax.jit
def kernel(x, token_table, pos_table):
    x_flat = x.reshape(-1).astype(jnp.int32)
    tok_lin = jex_layout.with_layout_constraint(
        token_table,
        jex_layout.Layout(major_to_minor=(0, 1), tiling=()))
    mesh = plsc.VectorSubcoreMesh(core_axis_name="c", subcore_axis_name="s")
    out = pl.kernel(
        _sc_body,
        out_type=jax.ShapeDtypeStruct((BATCH * MAX_SEQ, DIM), jnp.float32),
        mesh=mesh,
        scratch_types=[
            pltpu.VMEM((MAX_SEQ,), jnp.int32),
            pltpu.VMEM((CHUNK, DIM), jnp.float32),
            pltpu.VMEM((CHUNK, DIM), jnp.float32),
            pltpu.SemaphoreType.DMA,
        ],
        compiler_params=pltpu.CompilerParams(use_tc_tiling_on_sc=False),
    )(x_flat, tok_lin, pos_table)
    return out.reshape(BATCH, MAX_SEQ, DIM)
